# Initial kernel scaffold; baseline (speedup 1.0000x reference)
import jax
import jax.numpy as jnp
from jax.experimental import pallas as pl


def kernel(x):
    raise NotImplementedError("write your pallas kernel here")


# speedup vs baseline: 2.6105x; 2.6105x over previous
import functools

import jax
import jax.numpy as jnp
from jax.experimental import pallas as pl

K = 8
T = 128
S = 3 * T
J = K * K * K


def _tc_body(x_ref, o_ref):
    x = x_ref[0, 0]

    tp = jax.lax.broadcasted_iota(jnp.int32, (T, S), 0)
    s = jax.lax.broadcasted_iota(jnp.int32, (T, S), 1)
    src = s // 3 + s % 3 - 1
    m = (src == tp).astype(jnp.float32)
    y = jnp.dot(x, m, preferred_element_type=jnp.float32)

    y0 = jnp.broadcast_to(y.reshape(1, K, 1, S), (K, K, K, S)).reshape(J, S)
    y1 = jnp.broadcast_to(y.reshape(K, 1, 1, S), (K, K, K, S)).reshape(J, S)
    y2 = jnp.broadcast_to(y.reshape(1, 1, K, S), (K, K, K, S)).reshape(J, S)

    w = jax.lax.broadcasted_iota(jnp.int32, (J, S), 1) % 3
    o_ref[0, 0] = jnp.where(w == 0, y0, jnp.where(w == 1, y1, y2))


@jax.jit
def kernel(x):
    b, c = x.shape[0], x.shape[1]
    return pl.pallas_call(
        _tc_body,
        grid=(b, c),
        in_specs=[pl.BlockSpec((1, 1, K, T), lambda i, j: (i, j, 0, 0))],
        out_specs=pl.BlockSpec((1, 1, J, S), lambda i, j: (i, j, 0, 0)),
        out_shape=jax.ShapeDtypeStruct((b, c, J, S), jnp.float32),
    )(x)
